# Initial kernel scaffold; baseline (speedup 1.0000x reference)
import jax
import jax.numpy as jnp
from jax.experimental import pallas as pl


def kernel(teacher_logits, labels):
    raise NotImplementedError("write your pallas kernel here")


# speedup vs baseline: 1.0257x; 1.0257x over previous
import functools

import jax
import jax.numpy as jnp
import numpy as np
from jax import lax
from jax.experimental import pallas as pl
from jax.experimental.pallas import tpu as pltpu
from jax.experimental.pallas import tpu_sc as plsc

_B = 4096
_C = 100000
_K = 8
_NC = 2
_NS = 16
_NW = _NC * _NS
_RPW = _B // _NW
_CHUNK = 50000
_NCHUNK = _C // _CHUNK
_GROUP = 25
_NGROUP = _CHUNK // (16 * _GROUP)
_NEG = float("-inf")
_POS = float("inf")

_mesh = plsc.VectorSubcoreMesh(core_axis_name="c", subcore_axis_name="s",
                               num_cores=_NC, num_subcores=_NS)

_GATHER_DNUMS = lax.GatherDimensionNumbers(
    offset_dims=(), collapsed_slice_dims=(0,), start_index_map=(0,))


def _shuffle(x, idx):
    return lax.gather(x, idx[:, None], dimension_numbers=_GATHER_DNUMS,
                      slice_sizes=(1,),
                      mode=lax.GatherScatterMode.PROMISE_IN_BOUNDS)


def _checked_sort(lanes, k, v, descending):
    sk, sv = plsc.sort_key_val(k, v, descending=descending)
    nxt = jnp.minimum(lanes + 1, 15)
    tied = jnp.any((sk == _shuffle(sk, nxt)) & (lanes < 15))

    def exact(_):
        rank = jnp.zeros((16,), jnp.int32)
        for s in range(1, 16):
            rot = (lanes + s) & 15
            wk = _shuffle(k, rot)
            wv = _shuffle(v, rot)
            beats = (wk > k) | ((wk == k) & (wv < v))
            rank = rank + beats.astype(jnp.int32)
        _, perm = plsc.sort_key_val(rank, lanes, descending=not descending)
        return _shuffle(k, perm), _shuffle(v, perm)

    return lax.cond(tied, exact, lambda _: (sk, sv), jnp.int32(0))


@functools.partial(
    pl.kernel,
    mesh=_mesh,
    compiler_params=pltpu.CompilerParams(needs_layout_passes=False),
    out_type=(
        jax.ShapeDtypeStruct((_B * _K,), jnp.float32),
        jax.ShapeDtypeStruct((_B * _K,), jnp.int32),
    ),
    scratch_types=[
        pltpu.VMEM((_CHUNK,), jnp.float32),
        pltpu.VMEM((_CHUNK,), jnp.float32),
        pltpu.VMEM((_RPW * _K,), jnp.float32),
        pltpu.VMEM((_RPW * _K,), jnp.int32),
        pltpu.VMEM((_RPW,), jnp.int32),
        pltpu.SemaphoreType.DMA,
        pltpu.SemaphoreType.DMA,
    ],
)
def _mine_topk(logits_hbm, labels_hbm, outv_hbm, outi_hbm,
               chunk0_v, chunk1_v, outv_v, outi_v, lab_v, sem0, sem1):
    wid = lax.axis_index("s") * _NC + lax.axis_index("c")
    row0 = wid * _RPW
    lanes = lax.iota(jnp.int32, 16)
    low8 = lanes < 8
    last_off = np.int32(_B * _C - _CHUNK)

    pltpu.sync_copy(labels_hbm.at[pl.ds(pl.multiple_of(row0, 8), _RPW)], lab_v)

    def _start(off, buf, sem):
        pltpu.async_copy(
            logits_hbm.at[pl.ds(pl.multiple_of(off, 8), _CHUNK)], buf, sem)

    def _wait(buf, sem):
        pltpu.make_async_copy(
            logits_hbm.at[pl.ds(0, _CHUNK)], buf, sem).wait()

    def scan_chunk(buf, lab, c, carry):
        cbase = c * _CHUNK

        pos = lab - cbase
        in_rng = (pos >= 0) & (pos < _CHUNK)
        posc = jnp.clip(pos, 0, _CHUNK - 1)
        voff = pl.multiple_of((posc // 16) * 16, 16)
        lvec = buf[pl.ds(voff, 16)]
        hit = (lanes == (posc % 16)) & in_rng
        buf[pl.ds(voff, 16)] = jnp.where(hit, _NEG, lvec)

        def group_body(g, carry2):
            t8k, t8i, th = carry2
            gbase = pl.multiple_of(g * (_GROUP * 16), 16)
            vs = [buf[pl.ds(gbase + j * 16, 16)] for j in range(_GROUP)]
            while len(vs) > 1:
                nxt = [jnp.maximum(vs[i], vs[i + 1])
                       for i in range(0, len(vs) - 1, 2)]
                if len(vs) % 2:
                    nxt.append(vs[-1])
                vs = nxt
            gm = jnp.any(vs[0] > th)

            def rescan(c2):
                def vreg_body(j, c3):
                    t8k3, t8i3, th3 = c3
                    off = pl.multiple_of(gbase + j * 16, 16)
                    v = buf[pl.ds(off, 16)]
                    vm = jnp.any(v > th3)

                    def insert(c4):
                        t8k4, t8i4, th4 = c4
                        gidx = lanes + (cbase + off)
                        sk, sv = _checked_sort(lanes, v, gidx,
                                               descending=False)
                        mk = jnp.where(low8, t8k4, sk)
                        mi = jnp.where(low8, t8i4, sv)
                        sk2, sv2 = _checked_sort(lanes, mk, mi,
                                                 descending=True)
                        nk = jnp.where(low8, sk2, _NEG)
                        ni = jnp.where(low8, sv2, jnp.int32(0))
                        nth = jnp.zeros((16,), jnp.float32) + jnp.min(
                            jnp.where(low8, sk2, _POS))
                        return nk, ni, nth

                    return lax.cond(vm, insert, lambda c4: c4,
                                    (t8k3, t8i3, th3))

                return lax.fori_loop(0, _GROUP, vreg_body, c2)

            return lax.cond(gm, rescan, lambda c2: c2, (t8k, t8i, th))

        return lax.fori_loop(0, _NGROUP, group_body, carry)

    def scan_row(r):
        lv = lab_v[pl.ds(pl.multiple_of((r // 16) * 16, 16), 16)]
        lab = jnp.max(jnp.where(lanes == (r % 16),
                                lv.astype(jnp.float32),
                                jnp.float32(-1.0))).astype(jnp.int32)

        carry = (jnp.zeros((16,), jnp.float32) + _NEG,
                 jnp.zeros((16,), jnp.int32),
                 jnp.zeros((16,), jnp.float32) + _NEG)

        base = (row0 + r) * _C
        _start(base + _CHUNK, chunk1_v, sem1)
        _wait(chunk0_v, sem0)
        carry = scan_chunk(chunk0_v, lab, 0, carry)
        _start(jnp.minimum(base + _C, last_off), chunk0_v, sem0)
        _wait(chunk1_v, sem1)
        carry = scan_chunk(chunk1_v, lab, 1, carry)
        top8k, top8i, _unused = carry
        return top8k, top8i

    _start(row0 * _C, chunk0_v, sem0)

    def pair_body(p, pair_carry):
        k0, i0 = scan_row(p * 2)
        k1, i1 = scan_row(p * 2 + 1)
        shift8 = jnp.maximum(lanes - 8, 0)
        ck = jnp.where(low8, k0, _shuffle(k1, shift8))
        ci = jnp.where(low8, i0, _shuffle(i1, shift8))
        ooff = pl.multiple_of(p * 16, 16)
        outv_v[pl.ds(ooff, 16)] = ck
        outi_v[pl.ds(ooff, 16)] = ci
        return pair_carry

    lax.fori_loop(0, _RPW // 2, pair_body, jnp.int32(0))

    _wait(chunk0_v, sem0)

    pltpu.sync_copy(
        outv_v.at[pl.ds(0, _RPW * _K)],
        outv_hbm.at[pl.ds(pl.multiple_of(row0 * _K, 8), _RPW * _K)])
    pltpu.sync_copy(
        outi_v.at[pl.ds(0, _RPW * _K)],
        outi_hbm.at[pl.ds(pl.multiple_of(row0 * _K, 8), _RPW * _K)])


def kernel(teacher_logits, labels):
    flat = teacher_logits.reshape((_B * _C,))
    vals, idxs = _mine_topk(flat, labels.astype(jnp.int32))
    return vals.reshape(_B, _K), idxs.reshape(_B, _K)


# speedup vs baseline: 1.2256x; 1.1949x over previous
import functools

import jax
import jax.numpy as jnp
import numpy as np
from jax import lax
from jax.experimental import pallas as pl
from jax.experimental.pallas import tpu as pltpu
from jax.experimental.pallas import tpu_sc as plsc

_B = 4096
_C = 100000
_K = 8
_NC = 2
_NS = 16
_NW = _NC * _NS
_RPW = _B // _NW
_CHUNK = 50000
_NCHUNK = _C // _CHUNK
_GROUP = 25
_SUB = 5
_SUBN = _GROUP // _SUB
_NGROUP = _CHUNK // (16 * _GROUP)
_NEG = float("-inf")
_POS = float("inf")

_mesh = plsc.VectorSubcoreMesh(core_axis_name="c", subcore_axis_name="s",
                               num_cores=_NC, num_subcores=_NS)

_GATHER_DNUMS = lax.GatherDimensionNumbers(
    offset_dims=(), collapsed_slice_dims=(0,), start_index_map=(0,))


def _shuffle(x, idx):
    return lax.gather(x, idx[:, None], dimension_numbers=_GATHER_DNUMS,
                      slice_sizes=(1,),
                      mode=lax.GatherScatterMode.PROMISE_IN_BOUNDS)


def _checked_sort(lanes, k, v, descending):
    sk, sv = plsc.sort_key_val(k, v, descending=descending)
    nxt = jnp.minimum(lanes + 1, 15)
    tied = jnp.any((sk == _shuffle(sk, nxt)) & (lanes < 15))

    def exact(_):
        rank = jnp.zeros((16,), jnp.int32)
        for s in range(1, 16):
            rot = (lanes + s) & 15
            wk = _shuffle(k, rot)
            wv = _shuffle(v, rot)
            beats = (wk > k) | ((wk == k) & (wv < v))
            rank = rank + beats.astype(jnp.int32)
        _, perm = plsc.sort_key_val(rank, lanes, descending=not descending)
        return _shuffle(k, perm), _shuffle(v, perm)

    return lax.cond(tied, exact, lambda _: (sk, sv), jnp.int32(0))


@functools.partial(
    pl.kernel,
    mesh=_mesh,
    compiler_params=pltpu.CompilerParams(needs_layout_passes=False),
    out_type=(
        jax.ShapeDtypeStruct((_B * _K,), jnp.float32),
        jax.ShapeDtypeStruct((_B * _K,), jnp.int32),
    ),
    scratch_types=[
        pltpu.VMEM((_CHUNK,), jnp.float32),
        pltpu.VMEM((_CHUNK,), jnp.float32),
        pltpu.VMEM((_RPW * _K,), jnp.float32),
        pltpu.VMEM((_RPW * _K,), jnp.int32),
        pltpu.VMEM((_RPW,), jnp.int32),
        pltpu.SemaphoreType.DMA,
        pltpu.SemaphoreType.DMA,
    ],
)
def _mine_topk(logits_hbm, labels_hbm, outv_hbm, outi_hbm,
               chunk0_v, chunk1_v, outv_v, outi_v, lab_v, sem0, sem1):
    wid = lax.axis_index("s") * _NC + lax.axis_index("c")
    row0 = wid * _RPW
    lanes = lax.iota(jnp.int32, 16)
    low8 = lanes < 8
    last_off = np.int32(_B * _C - _CHUNK)

    pltpu.sync_copy(labels_hbm.at[pl.ds(pl.multiple_of(row0, 8), _RPW)], lab_v)

    def _start(off, buf, sem):
        pltpu.async_copy(
            logits_hbm.at[pl.ds(pl.multiple_of(off, 8), _CHUNK)], buf, sem)

    def _wait(buf, sem):
        pltpu.make_async_copy(
            logits_hbm.at[pl.ds(0, _CHUNK)], buf, sem).wait()

    def scan_chunk(buf, lab, c, carry):
        cbase = c * _CHUNK

        pos = lab - cbase
        in_rng = (pos >= 0) & (pos < _CHUNK)
        posc = jnp.clip(pos, 0, _CHUNK - 1)
        voff = pl.multiple_of((posc // 16) * 16, 16)
        lvec = buf[pl.ds(voff, 16)]
        hit = (lanes == (posc % 16)) & in_rng
        buf[pl.ds(voff, 16)] = jnp.where(hit, _NEG, lvec)

        def group_body(g, carry2):
            t8k, t8i, th = carry2
            gbase = pl.multiple_of(g * (_GROUP * 16), 16)
            vs = [buf[pl.ds(gbase + j * 16, 16)] for j in range(_GROUP)]
            while len(vs) > 1:
                nxt = [jnp.maximum(vs[i], vs[i + 1])
                       for i in range(0, len(vs) - 1, 2)]
                if len(vs) % 2:
                    nxt.append(vs[-1])
                vs = nxt
            gm = jnp.any(vs[0] > th)

            def rescan(c2):
                def vreg_body(sbase, j, c3):
                    t8k3, t8i3, th3 = c3
                    off = pl.multiple_of(sbase + j * 16, 16)
                    v = buf[pl.ds(off, 16)]
                    vm = jnp.any(v > th3)

                    def insert(c4):
                        t8k4, t8i4, th4 = c4
                        gidx = lanes + (cbase + off)
                        sk, sv = _checked_sort(lanes, v, gidx,
                                               descending=False)
                        mk = jnp.where(low8, t8k4, sk)
                        mi = jnp.where(low8, t8i4, sv)
                        sk2, sv2 = _checked_sort(lanes, mk, mi,
                                                 descending=True)
                        nk = jnp.where(low8, sk2, _NEG)
                        ni = jnp.where(low8, sv2, jnp.int32(0))
                        nth = jnp.zeros((16,), jnp.float32) + jnp.min(
                            jnp.where(low8, sk2, _POS))
                        return nk, ni, nth

                    return lax.cond(vm, insert, lambda c4: c4,
                                    (t8k3, t8i3, th3))

                def sub_body(sb, c3):
                    sbase = pl.multiple_of(gbase + sb * (_SUB * 16), 16)
                    svs = [buf[pl.ds(sbase + j * 16, 16)]
                           for j in range(_SUB)]
                    while len(svs) > 1:
                        nxt2 = [jnp.maximum(svs[i], svs[i + 1])
                                for i in range(0, len(svs) - 1, 2)]
                        if len(svs) % 2:
                            nxt2.append(svs[-1])
                        svs = nxt2
                    sgm = jnp.any(svs[0] > c3[2])

                    def sub_rescan(c4):
                        return lax.fori_loop(
                            0, _SUB,
                            lambda j, c5: vreg_body(sbase, j, c5), c4)

                    return lax.cond(sgm, sub_rescan, lambda c4: c4, c3)

                return lax.fori_loop(0, _SUBN, sub_body, c2)

            return lax.cond(gm, rescan, lambda c2: c2, (t8k, t8i, th))

        return lax.fori_loop(0, _NGROUP, group_body, carry)

    def scan_row(r):
        lv = lab_v[pl.ds(pl.multiple_of((r // 16) * 16, 16), 16)]
        lab = jnp.max(jnp.where(lanes == (r % 16),
                                lv.astype(jnp.float32),
                                jnp.float32(-1.0))).astype(jnp.int32)

        carry = (jnp.zeros((16,), jnp.float32) + _NEG,
                 jnp.zeros((16,), jnp.int32),
                 jnp.zeros((16,), jnp.float32) + _NEG)

        base = (row0 + r) * _C
        _start(base + _CHUNK, chunk1_v, sem1)
        _wait(chunk0_v, sem0)
        carry = scan_chunk(chunk0_v, lab, 0, carry)
        _start(jnp.minimum(base + _C, last_off), chunk0_v, sem0)
        _wait(chunk1_v, sem1)
        carry = scan_chunk(chunk1_v, lab, 1, carry)
        top8k, top8i, _unused = carry
        return top8k, top8i

    _start(row0 * _C, chunk0_v, sem0)

    def pair_body(p, pair_carry):
        k0, i0 = scan_row(p * 2)
        k1, i1 = scan_row(p * 2 + 1)
        shift8 = jnp.maximum(lanes - 8, 0)
        ck = jnp.where(low8, k0, _shuffle(k1, shift8))
        ci = jnp.where(low8, i0, _shuffle(i1, shift8))
        ooff = pl.multiple_of(p * 16, 16)
        outv_v[pl.ds(ooff, 16)] = ck
        outi_v[pl.ds(ooff, 16)] = ci
        return pair_carry

    lax.fori_loop(0, _RPW // 2, pair_body, jnp.int32(0))

    _wait(chunk0_v, sem0)

    pltpu.sync_copy(
        outv_v.at[pl.ds(0, _RPW * _K)],
        outv_hbm.at[pl.ds(pl.multiple_of(row0 * _K, 8), _RPW * _K)])
    pltpu.sync_copy(
        outi_v.at[pl.ds(0, _RPW * _K)],
        outi_hbm.at[pl.ds(pl.multiple_of(row0 * _K, 8), _RPW * _K)])


def kernel(teacher_logits, labels):
    flat = teacher_logits.reshape((_B * _C,))
    vals, idxs = _mine_topk(flat, labels.astype(jnp.int32))
    return vals.reshape(_B, _K), idxs.reshape(_B, _K)


# speedup vs baseline: 1.6546x; 1.3500x over previous
import functools

import jax
import jax.numpy as jnp
import numpy as np
from jax import lax
from jax.experimental import pallas as pl
from jax.experimental.pallas import tpu as pltpu
from jax.experimental.pallas import tpu_sc as plsc

_B = 4096
_C = 100000
_K = 8
_NC = 2
_NS = 16
_NW = _NC * _NS
_RPW = _B // _NW
_RG = 8
_NRG = _RPW // _RG
_CC = 6400
_TAILC = 32
_NCF = (_C - _TAILC) // _CC
_CLAST = _C - _TAILC - _NCF * _CC
_TSTART = _C - 128
if _CC % 400 == 0:
    _G1, _S1 = 25, [5, 5, 5, 5, 5]
else:
    _G1, _S1 = 4, [2, 2]
_NG1 = _CC // (16 * _G1)
_G2, _S2 = 31, [6, 6, 6, 6, 7]
_NG2 = _CLAST // (16 * _G2) if _CLAST else 0
_NEG = float("-inf")
_POS = float("inf")

_mesh = plsc.VectorSubcoreMesh(core_axis_name="c", subcore_axis_name="s",
                               num_cores=_NC, num_subcores=_NS)

_GATHER_DNUMS = lax.GatherDimensionNumbers(
    offset_dims=(), collapsed_slice_dims=(0,), start_index_map=(0,))


def _shuffle(x, idx):
    return lax.gather(x, idx[:, None], dimension_numbers=_GATHER_DNUMS,
                      slice_sizes=(1,),
                      mode=lax.GatherScatterMode.PROMISE_IN_BOUNDS)


def _checked_sort(lanes, k, v, descending):
    sk, sv = plsc.sort_key_val(k, v, descending=descending)
    nxt = jnp.minimum(lanes + 1, 15)
    tied = jnp.any((sk == _shuffle(sk, nxt)) & (lanes < 15))

    def exact(_):
        rank = jnp.zeros((16,), jnp.int32)
        for s in range(1, 16):
            rot = (lanes + s) & 15
            wk = _shuffle(k, rot)
            wv = _shuffle(v, rot)
            beats = (wk > k) | ((wk == k) & (wv < v))
            rank = rank + beats.astype(jnp.int32)
        _, perm = plsc.sort_key_val(rank, lanes, descending=not descending)
        return _shuffle(k, perm), _shuffle(v, perm)

    return lax.cond(tied, exact, lambda _: (sk, sv), jnp.int32(0))


@functools.partial(
    pl.kernel,
    mesh=_mesh,
    compiler_params=pltpu.CompilerParams(needs_layout_passes=False),
    out_type=(
        jax.ShapeDtypeStruct((_B * _K,), jnp.float32),
        jax.ShapeDtypeStruct((_B * _K,), jnp.int32),
    ),
    scratch_types=[
        pltpu.VMEM((_RG, _CC), jnp.float32),
        pltpu.VMEM((_RG, _CC), jnp.float32),
        pltpu.VMEM((_RPW, 128), jnp.float32),
        pltpu.VMEM((_RG, 16), jnp.float32),
        pltpu.VMEM((_RG, 16), jnp.int32),
        pltpu.VMEM((_RG, 16), jnp.float32),
        pltpu.VMEM((_RPW * _K,), jnp.float32),
        pltpu.VMEM((_RPW * _K,), jnp.int32),
        pltpu.VMEM((_RPW,), jnp.int32),
        pltpu.SemaphoreType.DMA,
        pltpu.SemaphoreType.DMA,
    ],
)
def _mine_topk(logits_hbm, tail_hbm, labels_hbm, outv_hbm, outi_hbm,
               buf0, buf1, tail_v, stk_v, sti_v, stt_v,
               outv_v, outi_v, lab_v, sem0, sem1):
    wid = lax.axis_index("s") * _NC + lax.axis_index("c")
    row0 = wid * _RPW
    lanes = lax.iota(jnp.int32, 16)
    low8 = lanes < 8

    pltpu.sync_copy(labels_hbm.at[pl.ds(pl.multiple_of(row0, 8), _RPW)], lab_v)
    pltpu.sync_copy(tail_hbm.at[pl.ds(pl.multiple_of(row0, 8), _RPW), :],
                    tail_v)

    def _startc(r0, col, buf, sem):
        pltpu.async_copy(
            logits_hbm.at[pl.ds(pl.multiple_of(r0, 8), _RG),
                          pl.ds(pl.multiple_of(col, 128), _CC)],
            buf, sem)

    def _waitc(buf, sem):
        pltpu.make_async_copy(
            logits_hbm.at[pl.ds(0, _RG), pl.ds(0, _CC)], buf, sem).wait()

    def _get_lab(i):
        lv = lab_v[pl.ds(pl.multiple_of((i // 16) * 16, 16), 16)]
        return jnp.max(jnp.where(lanes == (i % 16),
                                 lv.astype(jnp.float32),
                                 jnp.float32(-1.0))).astype(jnp.int32)

    def _insert(v, gidx_base, carry):
        t8k, t8i, th = carry
        gidx = lanes + gidx_base
        sk, sv = _checked_sort(lanes, v, gidx, descending=False)
        mk = jnp.where(low8, t8k, sk)
        mi = jnp.where(low8, t8i, sv)
        sk2, sv2 = _checked_sort(lanes, mk, mi, descending=True)
        nk = jnp.where(low8, sk2, _NEG)
        ni = jnp.where(low8, sv2, jnp.int32(0))
        nth = jnp.zeros((16,), jnp.float32) + jnp.min(
            jnp.where(low8, sk2, _POS))
        return nk, ni, nth

    def _tree(vecs):
        vs = list(vecs)
        while len(vs) > 1:
            nxt = [jnp.maximum(vs[i], vs[i + 1])
                   for i in range(0, len(vs) - 1, 2)]
            if len(vs) % 2:
                nxt.append(vs[-1])
            vs = nxt
        return vs[0]

    def _scan_rows(buf, rg, c0, group, ngroups, subs):
        def row_body(i, rc):
            lab = _get_lab(rg * _RG + i)
            t8k = stk_v[i, pl.ds(0, 16)]
            t8i = sti_v[i, pl.ds(0, 16)]
            th = stt_v[i, pl.ds(0, 16)]

            pos = lab - c0
            width = 16 * group * ngroups
            in_rng = (pos >= 0) & (pos < width)
            posc = jnp.clip(pos, 0, width - 1)
            voff = pl.multiple_of((posc // 16) * 16, 16)
            lvec = buf[i, pl.ds(voff, 16)]
            hit = (lanes == (posc % 16)) & in_rng
            buf[i, pl.ds(voff, 16)] = jnp.where(hit, _NEG, lvec)

            def group_body(g, carry):
                gbase = pl.multiple_of(g * (group * 16), 16)
                gm = jnp.any(_tree(buf[i, pl.ds(gbase + j * 16, 16)]
                                   for j in range(group)) > carry[2])

                def rescan(c2):
                    cc = c2
                    soff = 0
                    for sub in subs:
                        sbase = pl.multiple_of(gbase + soff * 16, 16)
                        sm = jnp.any(_tree(
                            buf[i, pl.ds(sbase + j * 16, 16)]
                            for j in range(sub)) > cc[2])

                        def sub_rescan(c3, sbase=sbase, sub=sub):
                            def vreg_body(j, c4):
                                off = pl.multiple_of(sbase + j * 16, 16)
                                v = buf[i, pl.ds(off, 16)]
                                vm = jnp.any(v > c4[2])
                                return lax.cond(
                                    vm,
                                    lambda c5: _insert(v, c0 + off, c5),
                                    lambda c5: c5, c4)

                            return lax.fori_loop(0, sub, vreg_body, c3)

                        cc = lax.cond(sm, sub_rescan, lambda c3: c3, cc)
                        soff += sub
                    return cc

                return lax.cond(gm, rescan, lambda c2: c2, carry)

            t8k, t8i, th = lax.fori_loop(0, ngroups, group_body,
                                         (t8k, t8i, th))
            stk_v[i, pl.ds(0, 16)] = t8k
            sti_v[i, pl.ds(0, 16)] = t8i
            stt_v[i, pl.ds(0, 16)] = th
            return rc

        lax.fori_loop(0, _RG, row_body, jnp.int32(0))

    def rg_body(rg, rgc):
        r0 = row0 + rg * _RG
        def init_body(i, ic):
            stk_v[i, pl.ds(0, 16)] = jnp.zeros((16,), jnp.float32) + _NEG
            sti_v[i, pl.ds(0, 16)] = jnp.zeros((16,), jnp.int32)
            stt_v[i, pl.ds(0, 16)] = jnp.zeros((16,), jnp.float32) + _NEG
            return ic
        lax.fori_loop(0, _RG, init_body, jnp.int32(0))

        _startc(r0, 0, buf0, sem0)

        def chunk_body(c, cc):
            nxt_col = jnp.minimum((c + 1) * _CC, _NCF * _CC - _CC)

            def even_branch(_):
                @pl.when(c + 1 < _NCF)
                def _():
                    _startc(r0, nxt_col, buf1, sem1)
                _waitc(buf0, sem0)
                _scan_rows(buf0, rg, c * _CC, _G1, _NG1, _S1)
                return jnp.int32(0)

            def odd_branch(_):
                @pl.when(c + 1 < _NCF)
                def _():
                    _startc(r0, nxt_col, buf0, sem0)
                _waitc(buf1, sem1)
                _scan_rows(buf1, rg, c * _CC, _G1, _NG1, _S1)
                return jnp.int32(0)

            lax.cond(c % 2 == 0, even_branch, odd_branch, jnp.int32(0))
            return cc

        lax.fori_loop(0, _NCF, chunk_body, jnp.int32(0))

        if _CLAST:
            pltpu.sync_copy(
                logits_hbm.at[pl.ds(pl.multiple_of(r0, 8), _RG),
                              pl.ds(_NCF * _CC, _CLAST)],
                buf0.at[:, pl.ds(0, _CLAST)])
            _scan_rows(buf0, rg, _NCF * _CC, _G2, _NG2, _S2)

        def tail_body(i, tc):
            lab = _get_lab(rg * _RG + i)
            carry = (stk_v[i, pl.ds(0, 16)], sti_v[i, pl.ds(0, 16)],
                     stt_v[i, pl.ds(0, 16)])
            for tv in range(6, 8):
                row = rg * _RG + i
                v = tail_v[row, pl.ds(tv * 16, 16)]
                gbase = _TSTART + tv * 16
                hit = (lanes + gbase) == lab
                v = jnp.where(hit, _NEG, v)
                vm = jnp.any(v > carry[2])
                carry = lax.cond(
                    vm, lambda c5, v=v, gbase=gbase: _insert(v, gbase, c5),
                    lambda c5: c5, carry)
            t8k, t8i, _unused = carry
            shift8 = jnp.maximum(lanes - 8, 0)
            obase = pl.multiple_of(((rg * _RG + i) // 2) * 16, 16)
            prev_k = outv_v[pl.ds(obase, 16)]
            prev_i = outi_v[pl.ds(obase, 16)]
            is_odd = (i % 2) == 1
            ck = jnp.where(low8, jnp.where(is_odd, prev_k, t8k),
                           jnp.where(is_odd, _shuffle(t8k, shift8), prev_k))
            ci = jnp.where(low8, jnp.where(is_odd, prev_i, t8i),
                           jnp.where(is_odd, _shuffle(t8i, shift8), prev_i))
            outv_v[pl.ds(obase, 16)] = ck
            outi_v[pl.ds(obase, 16)] = ci
            return tc

        lax.fori_loop(0, _RG, tail_body, jnp.int32(0))
        return rgc

    lax.fori_loop(0, _NRG, rg_body, jnp.int32(0))

    pltpu.sync_copy(
        outv_v.at[pl.ds(0, _RPW * _K)],
        outv_hbm.at[pl.ds(pl.multiple_of(row0 * _K, 8), _RPW * _K)])
    pltpu.sync_copy(
        outi_v.at[pl.ds(0, _RPW * _K)],
        outi_hbm.at[pl.ds(pl.multiple_of(row0 * _K, 8), _RPW * _K)])


def kernel(teacher_logits, labels):
    tail = lax.slice(teacher_logits, (0, _TSTART), (_B, _C))
    vals, idxs = _mine_topk(teacher_logits, tail, labels.astype(jnp.int32))
    return vals.reshape(_B, _K), idxs.reshape(_B, _K)
